# baseline (speedup 1.0000x reference)
import jax
import jax.numpy as jnp
from jax import lax
from jax.experimental import pallas as pl
from jax.experimental.pallas import tpu as pltpu
from jax.experimental.pallas import tpu_sc as plsc

B, C, H, W = 32, 2, 128, 128
HW = H * W
K = 500
KP = 512
GCH = 128
EPS = 1e-7
_TAN_3PI_8 = 2.414213562373095
_TAN_PI_8 = 0.414213562373095


def _atanq16(a, b):
    aa = jnp.abs(a)
    ab = jnp.abs(b)
    c1 = aa > _TAN_3PI_8 * ab
    c2 = aa > _TAN_PI_8 * ab
    num = jnp.where(c1, -ab, jnp.where(c2, aa - ab, aa))
    den = jnp.where(c1, aa, jnp.where(c2, aa + ab, ab))
    q = num / den
    y0 = jnp.where(c1, jnp.full_like(aa, jnp.pi / 2),
                   jnp.where(c2, jnp.full_like(aa, jnp.pi / 4),
                             jnp.zeros_like(aa)))
    z = q * q
    p = ((8.05374449538e-2 * z - 1.38776856032e-1) * z + 1.99777106478e-1) * z - 3.33329491539e-1
    r = y0 + (p * z * q + q)
    return jnp.where((a < 0) ^ (b < 0), -r, r)


def _ciou16(px, py, pw, ph, tx, ty, tw, th):
    p_x1 = px - pw * 0.5
    p_x2 = px + pw * 0.5
    p_y1 = py - ph * 0.5
    p_y2 = py + ph * 0.5
    t_x1 = tx - tw * 0.5
    t_x2 = tx + tw * 0.5
    t_y1 = ty - th * 0.5
    t_y2 = ty + th * 0.5
    iw = jnp.maximum(jnp.minimum(p_x2, t_x2) - jnp.maximum(p_x1, t_x1), 0.0)
    ih = jnp.maximum(jnp.minimum(p_y2, t_y2) - jnp.maximum(p_y1, t_y1), 0.0)
    inter = iw * ih
    union = pw * ph + tw * th - inter + EPS
    cw = jnp.maximum(p_x2, t_x2) - jnp.minimum(p_x1, t_x1)
    ch = jnp.maximum(p_y2, t_y2) - jnp.minimum(p_y1, t_y1)
    c2 = cw * cw + ch * ch + EPS
    dx = px - tx
    dy = py - ty
    rho2 = dx * dx + dy * dy
    da = _atanq16(tw, th + EPS) - _atanq16(pw, ph + EPS)
    v = (4.0 / (jnp.pi ** 2)) * (da * da)
    d = (1.0 + v + EPS) * union - inter
    uc = union * c2
    numer = inter * c2 * d - rho2 * union * d - (v * v) * union * uc
    return numer / (uc * d)


def kernel(output_wh, output_reg, mask, ind, target_wh, target_reg):
    wh_flat = output_wh.reshape(B * C * HW)
    reg_flat = output_reg.reshape(B * C * HW)
    pad2 = ((0, 0), (0, KP - K))
    pad3 = ((0, 0), (0, KP - K), (0, 0))
    ind_p = jnp.pad(ind, pad2).reshape(16, 2, KP)
    mask_p = jnp.pad(mask, pad2).reshape(16, 2, KP)
    im = jnp.stack([ind_p, mask_p], axis=2).reshape(16, 2 * 2 * KP)
    tw_ = jnp.pad(target_wh, pad3)
    tr_ = jnp.pad(target_reg, pad3)
    tg = jnp.stack([tw_[:, :, 0].reshape(16, 2, KP), tw_[:, :, 1].reshape(16, 2, KP),
                    tr_[:, :, 0].reshape(16, 2, KP), tr_[:, :, 1].reshape(16, 2, KP)],
                   axis=2).reshape(16, 2 * 4 * KP)
    pk = jnp.concatenate([im.astype(jnp.float32), tg], axis=1)

    mesh = plsc.VectorSubcoreMesh(core_axis_name="c", subcore_axis_name="s",
                                  num_cores=1)

    def body(wh_hbm, reg_hbm, pk_hbm, out_hbm,
             pk_v, rows0_v, rows1_v,
             pwh0_v, pwh1_v, preg0_v, preg1_v,
             part_v, tmp_v, shared, sem0, sem1):
        s = lax.axis_index("s")
        zero16f = jnp.zeros((16,), jnp.float32)

        pltpu.sync_copy(pk_hbm.at[s, pl.ds(0, 2048)], pk_v.at[pl.ds(0, 2048)])
        cps = {0: [pltpu.async_copy(pk_hbm.at[s, pl.ds(2048, 4096)],
                                    pk_v.at[pl.ds(2048, 4096)], sem0)], 1: []}

        for h in (0, 1):
            base = (2 * s + h) * (C * HW)

            def rows_body(j, carry, h=h, base=base):
                ich = pk_v[pl.ds(h * 1024 + j * 16, 16)].astype(jnp.int32)
                r0 = ich + base
                rows0_v[pl.ds(h * KP + j * 16, 16)] = r0
                rows1_v[pl.ds(h * KP + j * 16, 16)] = r0 + HW
                return carry

            lax.fori_loop(0, KP // 16, rows_body, 0, unroll=4)

            sem = sem0 if h == 0 else sem1
            for g in range(KP // GCH):
                sl = pl.ds(h * KP + g * GCH, GCH)
                cps[h].append(pltpu.async_copy(wh_hbm.at[rows0_v.at[sl]], pwh0_v.at[sl], sem))
                cps[h].append(pltpu.async_copy(wh_hbm.at[rows1_v.at[sl]], pwh1_v.at[sl], sem))
                cps[h].append(pltpu.async_copy(reg_hbm.at[rows0_v.at[sl]], preg0_v.at[sl], sem))
                cps[h].append(pltpu.async_copy(reg_hbm.at[rows1_v.at[sl]], preg1_v.at[sl], sem))

        acc = (zero16f, zero16f)
        for h in (0, 1):
            for cp in cps[h]:
                cp.wait()

            def comp(j, carry, h=h):
                al, ac = carry
                o = j * 16
                ich = pk_v[pl.ds(h * 1024 + o, 16)].astype(jnp.int32)
                mf = pk_v[pl.ds(h * 1024 + KP + o, 16)]
                xi = (ich & (W - 1)).astype(jnp.float32)
                yi = lax.shift_right_logical(ich, 7).astype(jnp.float32)
                pw = pwh0_v[pl.ds(h * KP + o, 16)]
                ph = pwh1_v[pl.ds(h * KP + o, 16)]
                px = xi + preg0_v[pl.ds(h * KP + o, 16)]
                py = yi + preg1_v[pl.ds(h * KP + o, 16)]
                tbase = 2048 + h * 4 * KP
                tw = pk_v[pl.ds(tbase + o, 16)] * 0.5
                th = pk_v[pl.ds(tbase + KP + o, 16)] * 0.5
                tx = xi + pk_v[pl.ds(tbase + 2 * KP + o, 16)]
                ty = yi + pk_v[pl.ds(tbase + 3 * KP + o, 16)]
                ciou = _ciou16(px, py, pw, ph, tx, ty, tw, th)
                al = al + jnp.where(mf > 0, 1.0 - ciou, 0.0)
                ac = ac + mf
                return (al, ac)

            acc = lax.fori_loop(0, KP // 16, comp, acc, unroll=4)
        acc_l, acc_c = acc

        part_v[pl.ds(0, 16)] = acc_l
        part_v[pl.ds(16, 16)] = acc_c
        pltpu.sync_copy(part_v, shared.at[pl.ds(s * 32, 32)])
        plsc.subcore_barrier()

        @pl.when(s == 0)
        def _():
            pltpu.sync_copy(shared, tmp_v)

            def red(t, carry):
                al, ac = carry
                return (al + tmp_v[pl.ds(t * 32, 16)],
                        ac + tmp_v[pl.ds(t * 32 + 16, 16)])

            al, ac = lax.fori_loop(0, 16, red, (zero16f, zero16f))

            def lanesum(v):
                for k in (8, 4, 2, 1):
                    perm = (lax.iota(jnp.int32, 16) + k) & 15
                    v = v + v.at[perm].get(mode="promise_in_bounds")
                return v

            part_v[pl.ds(0, 16)] = lanesum(al) / (lanesum(ac) + 1e-4)
            pltpu.sync_copy(part_v.at[pl.ds(0, 1)], out_hbm)

    out = pl.kernel(
        body,
        out_type=jax.ShapeDtypeStruct((1,), jnp.float32),
        mesh=mesh,
        scratch_types=[
            pltpu.VMEM((6 * 1024,), jnp.float32),
            pltpu.VMEM((2 * KP,), jnp.int32),
            pltpu.VMEM((2 * KP,), jnp.int32),
            pltpu.VMEM((2 * KP,), jnp.float32),
            pltpu.VMEM((2 * KP,), jnp.float32),
            pltpu.VMEM((2 * KP,), jnp.float32),
            pltpu.VMEM((2 * KP,), jnp.float32),
            pltpu.VMEM((32,), jnp.float32),
            pltpu.VMEM((512,), jnp.float32),
            pltpu.VMEM_SHARED((512,), jnp.float32),
            pltpu.SemaphoreType.DMA,
            pltpu.SemaphoreType.DMA,
        ],
    )(wh_flat, reg_flat, pk)
    return out.reshape(())


# speedup vs baseline: 1.0373x; 1.0373x over previous
import jax
import jax.numpy as jnp
from jax import lax
from jax.experimental import pallas as pl
from jax.experimental.pallas import tpu as pltpu
from jax.experimental.pallas import tpu_sc as plsc

B, C, H, W = 32, 2, 128, 128
HW = H * W
K = 500
KP = 512
GCH = 128
EPS = 1e-7
_TAN_3PI_8 = 2.414213562373095
_TAN_PI_8 = 0.414213562373095


def _atanq16(a, b):
    aa = jnp.abs(a)
    ab = jnp.abs(b)
    c1 = aa > _TAN_3PI_8 * ab
    c2 = aa > _TAN_PI_8 * ab
    num = jnp.where(c1, -ab, jnp.where(c2, aa - ab, aa))
    den = jnp.where(c1, aa, jnp.where(c2, aa + ab, ab))
    q = num / den
    y0 = jnp.where(c1, jnp.full_like(aa, jnp.pi / 2),
                   jnp.where(c2, jnp.full_like(aa, jnp.pi / 4),
                             jnp.zeros_like(aa)))
    z = q * q
    p = ((8.05374449538e-2 * z - 1.38776856032e-1) * z + 1.99777106478e-1) * z - 3.33329491539e-1
    r = y0 + (p * z * q + q)
    return jnp.where((a < 0) ^ (b < 0), -r, r)


def _ciou16(px, py, pw, ph, tx, ty, tw, th):
    p_x1 = px - pw * 0.5
    p_x2 = px + pw * 0.5
    p_y1 = py - ph * 0.5
    p_y2 = py + ph * 0.5
    t_x1 = tx - tw * 0.5
    t_x2 = tx + tw * 0.5
    t_y1 = ty - th * 0.5
    t_y2 = ty + th * 0.5
    iw = jnp.maximum(jnp.minimum(p_x2, t_x2) - jnp.maximum(p_x1, t_x1), 0.0)
    ih = jnp.maximum(jnp.minimum(p_y2, t_y2) - jnp.maximum(p_y1, t_y1), 0.0)
    inter = iw * ih
    union = pw * ph + tw * th - inter + EPS
    cw = jnp.maximum(p_x2, t_x2) - jnp.minimum(p_x1, t_x1)
    ch = jnp.maximum(p_y2, t_y2) - jnp.minimum(p_y1, t_y1)
    c2 = cw * cw + ch * ch + EPS
    dx = px - tx
    dy = py - ty
    rho2 = dx * dx + dy * dy
    da = _atanq16(tw, th + EPS) - _atanq16(pw, ph + EPS)
    v = (4.0 / (jnp.pi ** 2)) * (da * da)
    d = (1.0 + v + EPS) * union - inter
    uc = union * c2
    numer = inter * c2 * d - rho2 * union * d - (v * v) * union * uc
    return numer / (uc * d)


def kernel(output_wh, output_reg, mask, ind, target_wh, target_reg):
    wh_flat = output_wh.reshape(B * C * HW)
    reg_flat = output_reg.reshape(B * C * HW)
    pad2 = ((0, 0), (0, KP - K))
    pad3 = ((0, 0), (0, KP - K), (0, 0))
    ind_p = jnp.pad(ind, pad2).reshape(16, 2, KP)
    mask_p = jnp.pad(mask, pad2).reshape(16, 2, KP)
    im = jnp.stack([ind_p, mask_p], axis=2).reshape(16, 2 * 2 * KP)
    tw_ = jnp.pad(target_wh, pad3) * 0.5
    tr_ = jnp.pad(target_reg, pad3)
    tg = jnp.stack([tw_[:, :, 0].reshape(16, 2, KP), tw_[:, :, 1].reshape(16, 2, KP),
                    tr_[:, :, 0].reshape(16, 2, KP), tr_[:, :, 1].reshape(16, 2, KP)],
                   axis=2).reshape(16, 2 * 4 * KP)
    pk = jnp.concatenate([im.astype(jnp.float32), tg], axis=1)

    mesh = plsc.VectorSubcoreMesh(core_axis_name="c", subcore_axis_name="s",
                                  num_cores=1)

    def body(wh_hbm, reg_hbm, pk_hbm, out_hbm,
             pk_v, rows0_v, rows1_v,
             pwh0_v, pwh1_v, preg0_v, preg1_v,
             part_v, tmp_v, shared, sem0, sem1):
        s = lax.axis_index("s")
        zero16f = jnp.zeros((16,), jnp.float32)

        pltpu.sync_copy(pk_hbm.at[s, pl.ds(0, 2048)], pk_v.at[pl.ds(0, 2048)])
        cps = {0: [pltpu.async_copy(pk_hbm.at[s, pl.ds(2048, 4096)],
                                    pk_v.at[pl.ds(2048, 4096)], sem0)], 1: []}

        for h in (0, 1):
            base = (2 * s + h) * (C * HW)

            def rows_body(j, carry, h=h, base=base):
                ich = pk_v[pl.ds(h * 1024 + j * 16, 16)].astype(jnp.int32)
                r0 = ich + base
                rows0_v[pl.ds(h * KP + j * 16, 16)] = r0
                rows1_v[pl.ds(h * KP + j * 16, 16)] = r0 + HW
                return carry

            lax.fori_loop(0, KP // 16, rows_body, 0, unroll=4)

            sem = sem0 if h == 0 else sem1
            for g in range(KP // GCH):
                sl = pl.ds(h * KP + g * GCH, GCH)
                cps[h].append(pltpu.async_copy(wh_hbm.at[rows0_v.at[sl]], pwh0_v.at[sl], sem))
                cps[h].append(pltpu.async_copy(wh_hbm.at[rows1_v.at[sl]], pwh1_v.at[sl], sem))
                cps[h].append(pltpu.async_copy(reg_hbm.at[rows0_v.at[sl]], preg0_v.at[sl], sem))
                cps[h].append(pltpu.async_copy(reg_hbm.at[rows1_v.at[sl]], preg1_v.at[sl], sem))

        acc = (zero16f, zero16f)
        for h in (0, 1):
            for cp in cps[h]:
                cp.wait()

            def comp(j, carry, h=h):
                al, ac = carry
                o = j * 16
                ich = pk_v[pl.ds(h * 1024 + o, 16)].astype(jnp.int32)
                mf = pk_v[pl.ds(h * 1024 + KP + o, 16)]
                xi = (ich & (W - 1)).astype(jnp.float32)
                yi = lax.shift_right_logical(ich, 7).astype(jnp.float32)
                pw = pwh0_v[pl.ds(h * KP + o, 16)]
                ph = pwh1_v[pl.ds(h * KP + o, 16)]
                px = xi + preg0_v[pl.ds(h * KP + o, 16)]
                py = yi + preg1_v[pl.ds(h * KP + o, 16)]
                tbase = 2048 + h * 4 * KP
                tw = pk_v[pl.ds(tbase + o, 16)]
                th = pk_v[pl.ds(tbase + KP + o, 16)]
                tx = xi + pk_v[pl.ds(tbase + 2 * KP + o, 16)]
                ty = yi + pk_v[pl.ds(tbase + 3 * KP + o, 16)]
                ciou = _ciou16(px, py, pw, ph, tx, ty, tw, th)
                al = al + jnp.where(mf > 0, 1.0 - ciou, 0.0)
                ac = ac + mf
                return (al, ac)

            acc = lax.fori_loop(0, KP // 16, comp, acc, unroll=2)
        acc_l, acc_c = acc

        part_v[pl.ds(0, 16)] = acc_l
        part_v[pl.ds(16, 16)] = acc_c
        pltpu.sync_copy(part_v, shared.at[pl.ds(s * 32, 32)])
        plsc.subcore_barrier()

        @pl.when(s == 0)
        def _():
            pltpu.sync_copy(shared, tmp_v)

            al, ac = zero16f, zero16f
            for t in range(16):
                al = al + tmp_v[pl.ds(t * 32, 16)]
                ac = ac + tmp_v[pl.ds(t * 32 + 16, 16)]

            def lanesum(v):
                for k in (8, 4, 2, 1):
                    perm = (lax.iota(jnp.int32, 16) + k) & 15
                    v = v + v.at[perm].get(mode="promise_in_bounds")
                return v

            part_v[pl.ds(0, 16)] = lanesum(al) / (lanesum(ac) + 1e-4)
            pltpu.sync_copy(part_v.at[pl.ds(0, 1)], out_hbm)

    out = pl.kernel(
        body,
        out_type=jax.ShapeDtypeStruct((1,), jnp.float32),
        mesh=mesh,
        scratch_types=[
            pltpu.VMEM((6 * 1024,), jnp.float32),
            pltpu.VMEM((2 * KP,), jnp.int32),
            pltpu.VMEM((2 * KP,), jnp.int32),
            pltpu.VMEM((2 * KP,), jnp.float32),
            pltpu.VMEM((2 * KP,), jnp.float32),
            pltpu.VMEM((2 * KP,), jnp.float32),
            pltpu.VMEM((2 * KP,), jnp.float32),
            pltpu.VMEM((32,), jnp.float32),
            pltpu.VMEM((512,), jnp.float32),
            pltpu.VMEM_SHARED((512,), jnp.float32),
            pltpu.SemaphoreType.DMA,
            pltpu.SemaphoreType.DMA,
        ],
    )(wh_flat, reg_flat, pk)
    return out.reshape(())
